# Initial kernel scaffold; baseline (speedup 1.0000x reference)
import jax
import jax.numpy as jnp
from jax.experimental import pallas as pl


def kernel(x, w_up1, s_up1, b_up1, w_ra, s_ra, b_ra, w_rb_hwio, s_rb, b_rb, w_bot, b_bot, s_tie, b_tie, w_tie):
    raise NotImplementedError("write your pallas kernel here")


# speedup vs baseline: 1.0746x; 1.0746x over previous
import functools

import jax
import jax.numpy as jnp
from jax import lax
from jax.experimental import pallas as pl
from jax.experimental.pallas import tpu as pltpu


def _silu(v):
    return v * jax.nn.sigmoid(v)


def _csp1_body(C_, H, W, n_res,
               x_ref, wub_ref, wra_ref, wrb_ref, wtie_ref, sb_ref,
               out_ref):
    M = H * W

    sb = sb_ref[...]
    b_ub = sb[0 * C_:2 * C_]
    bra = sb[2 * C_:3 * C_]
    brb = sb[3 * C_:4 * C_]
    stie_u = sb[4 * C_:5 * C_]
    btie_u = sb[5 * C_:6 * C_]

    x = x_ref[...].astype(jnp.bfloat16)

    yb = jnp.dot(wub_ref[...], x, preferred_element_type=jnp.float32) + b_ub
    y = _silu(yb[:C_])
    bot_tt = yb[C_:]

    pix = lax.broadcasted_iota(jnp.int32, (1, M), 1)
    i_idx = pix // W
    j_idx = pix % W
    row_ok = {-1: i_idx >= 1, 1: i_idx < (H - 1)}
    col_ok = {-1: j_idx >= 1, 1: j_idx < (W - 1)}
    tap_mask = {}
    for r in (-1, 0, 1):
        for c in (-1, 0, 1):
            if r == 0 and c == 0:
                msk = None
            elif r == 0:
                msk = col_ok[c]
            elif c == 0:
                msk = row_ok[r]
            else:
                msk = row_ok[r] & col_ok[c]
            tap_mask[(r, c)] = msk

    zero_bf = jnp.zeros((), jnp.bfloat16)
    cur = y
    for _ in range(n_res):
        t = _silu(jnp.dot(wra_ref[...], cur.astype(jnp.bfloat16),
                          preferred_element_type=jnp.float32) + bra)
        tb = t.astype(jnp.bfloat16)
        taps = []
        for r in (-1, 0, 1):
            for c in (-1, 0, 1):
                off = r * W + c
                sh = tb if off == 0 else pltpu.roll(tb, shift=(-off) % M, axis=1)
                msk = tap_mask[(r, c)]
                if msk is not None:
                    sh = jnp.where(msk, sh, zero_bf)
                taps.append(sh)
        col = jnp.concatenate(taps, axis=0)
        cur = _silu(jnp.dot(wrb_ref[...], col,
                            preferred_element_type=jnp.float32) + brb)
    up = y + cur

    tt = jnp.concatenate([up * stie_u + btie_u, bot_tt], axis=0)
    tt = jnp.where(tt >= 0, tt, 0.01 * tt).astype(jnp.bfloat16)
    out_ref[...] = jnp.dot(wtie_ref[...], tt, preferred_element_type=jnp.float32)


def _full_spec(shape):
    nd = len(shape)
    return pl.BlockSpec(shape, lambda n, _nd=nd: (0,) * _nd)


def kernel(x, w_up1, s_up1, b_up1, w_ra, s_ra, b_ra, w_rb_hwio, s_rb, b_rb,
           w_bot, b_bot, s_tie, b_tie, w_tie):
    n_res = 2
    N, C1, H, W = x.shape
    C_ = w_up1.shape[1]
    C2 = w_tie.shape[1]
    M = H * W

    xf = x.reshape(N, C1, M)

    s_up1 = s_up1.reshape(-1); b_up1 = b_up1.reshape(-1)
    s_ra = s_ra.reshape(-1); b_ra = b_ra.reshape(-1)
    s_rb = s_rb.reshape(-1); b_rb = b_rb.reshape(-1)
    b_bot = b_bot.reshape(-1)
    s_tie = s_tie.reshape(-1); b_tie = b_tie.reshape(-1)
    s_tie_u, s_tie_b = s_tie[:C_], s_tie[C_:]
    b_tie_u, b_tie_b = b_tie[:C_], b_tie[C_:]

    wu = w_up1.T * s_up1[:, None]
    wbot = w_bot.T * s_tie_b[:, None]
    w_ub = jnp.concatenate([wu, wbot], axis=0).astype(jnp.bfloat16)
    wra = (w_ra.T * s_ra[:, None]).astype(jnp.bfloat16)
    wrb = (jnp.transpose(w_rb_hwio, (3, 0, 1, 2)).reshape(C_, 9 * C_)
           * s_rb[:, None]).astype(jnp.bfloat16)
    wtie = w_tie.T.astype(jnp.bfloat16)

    b_bot_f = s_tie_b * b_bot + b_tie_b
    sb = jnp.concatenate([b_up1, b_bot_f, b_ra, b_rb, s_tie_u, b_tie_u]
                         ).reshape(-1, 1).astype(jnp.float32)

    body = functools.partial(_csp1_body, C_, H, W, n_res)

    out = pl.pallas_call(
        body,
        out_shape=jax.ShapeDtypeStruct((N, C2, M), jnp.float32),
        grid=(N,),
        in_specs=[
            pl.BlockSpec((None, C1, M), lambda n: (n, 0, 0)),
            _full_spec(w_ub.shape), _full_spec(wra.shape), _full_spec(wrb.shape),
            _full_spec(wtie.shape), _full_spec(sb.shape),
        ],
        out_specs=pl.BlockSpec((None, C2, M), lambda n: (n, 0, 0)),
        compiler_params=pltpu.CompilerParams(
            dimension_semantics=("parallel",)),
    )(xf, w_ub, wra, wrb, wtie, sb)

    return out.reshape(N, C2, H, W)
